# baseline (speedup 1.0000x reference)
import functools

import jax
import jax.numpy as jnp
from jax import lax
from jax.experimental import pallas as pl
from jax.experimental.pallas import tpu as pltpu
from jax.experimental.pallas import tpu_sc as plsc

D = 64
W = 128
K = 20
UROWS = K + 1
NW = 32
CH = 32
DT = D // 16
CHA = 160

_GDN = lax.GatherDimensionNumbers(
    offset_dims=(), collapsed_slice_dims=(0,), start_index_map=(0,))


def _reg_gather(v, idx):
    return lax.gather(v, idx[:, None], _GDN, (1,),
                      mode=lax.GatherScatterMode.PROMISE_IN_BOUNDS)


def _hsum(v, perms):
    for p in perms:
        v = v + _reg_gather(v, p)
    return v


def _mesh():
    return plsc.VectorSubcoreMesh(core_axis_name="c", subcore_axis_name="s",
                                  num_cores=2, num_subcores=16)


def _sc_merge(emb_v, emb_u):
    R = emb_v.shape[0]
    ncha = R // CHA

    @functools.partial(
        pl.kernel,
        out_type=jax.ShapeDtypeStruct((R, W), jnp.float32),
        mesh=_mesh(),
        scratch_types=[
            pltpu.VMEM((CHA, D), jnp.float32),
            pltpu.VMEM((CHA, D), jnp.float32),
            pltpu.VMEM((CHA, W), jnp.float32),
        ],
        compiler_params=pltpu.CompilerParams(use_tc_tiling_on_sc=True),
    )
    def a(v_hbm, u_hbm, out_hbm, vb, ub, wide):
        wid = lax.axis_index("s") * 2 + lax.axis_index("c")
        nloop = (ncha + NW - 1) // NW

        def body(i, carry):
            chs = wid + i * NW

            @pl.when(chs < ncha)
            def _():
                r0 = chs * CHA
                pltpu.sync_copy(v_hbm.at[pl.ds(r0, CHA), :], vb)
                pltpu.sync_copy(u_hbm.at[pl.ds(r0, CHA), :], ub)

                def move(j, c2):
                    for t in range(DT):
                        wide[j, pl.ds(16 * t, 16)] = vb[j, pl.ds(16 * t, 16)]
                        wide[j, pl.ds(D + 16 * t, 16)] = \
                            ub[j, pl.ds(16 * t, 16)]
                    return c2

                lax.fori_loop(0, CHA, move, 0)
                pltpu.sync_copy(wide, out_hbm.at[pl.ds(r0, CHA), :])
            return carry

        lax.fori_loop(0, nloop, body, 0)

    return a(emb_v, emb_u)


def _sc_scores(cidx, uidx, tbl):
    B = cidx.shape[0]
    per_w = B // NW
    nch = per_w // CH

    @functools.partial(
        pl.kernel,
        out_type=[jax.ShapeDtypeStruct((B,), jnp.float32),
                  jax.ShapeDtypeStruct((B,), jnp.float32)],
        mesh=_mesh(),
        scratch_types=[
            pltpu.VMEM((CH,), jnp.int32),
            pltpu.VMEM((CH * UROWS,), jnp.int32),
            pltpu.VMEM((CH, W), jnp.float32),
            pltpu.VMEM((CH * UROWS, W), jnp.float32),
            pltpu.VMEM((CH,), jnp.float32),
            pltpu.VMEM((CH,), jnp.float32),
            pltpu.SemaphoreType.DMA,
        ],
        compiler_params=pltpu.CompilerParams(use_tc_tiling_on_sc=True),
    )
    def k(tbl_hbm, cidx_hbm, uidx_hbm, pos_hbm, neg_hbm,
          cidx_v, uidx_v, crow_v, urow_v, pos_v, neg_v, sem):
        wid = lax.axis_index("s") * 2 + lax.axis_index("c")
        base_w = wid * per_w
        lanes = lax.iota(jnp.int32, 16)
        perms = [lanes ^ s for s in (1, 2, 4, 8)]

        def chunk_body(ch, carry):
            base = base_w + ch * CH
            pltpu.sync_copy(cidx_hbm.at[pl.ds(base, CH)], cidx_v)
            pltpu.sync_copy(uidx_hbm.at[pl.ds(base * UROWS, CH * UROWS)],
                            uidx_v)
            copies = [pltpu.async_copy(tbl_hbm.at[cidx_v], crow_v, sem)]
            for r in range(UROWS):
                copies.append(pltpu.async_copy(
                    tbl_hbm.at[uidx_v.at[pl.ds(r * CH, CH)]],
                    urow_v.at[pl.ds(r * CH, CH)], sem))
            for cpy in copies:
                cpy.wait()

            zero16 = jnp.zeros((16,), jnp.float32)

            def group_body(g, c3):
                jbase = g * 16

                def item_body(l, acc):
                    accp, accn = acc
                    j = jbase + l
                    c = [crow_v[j, pl.ds(16 * t, 16)] for t in range(DT)]
                    ub = j * UROWS
                    tg = [urow_v[ub, pl.ds(D + 16 * t, 16)]
                          for t in range(DT)]
                    ap = c[0] * tg[0]
                    for t in range(1, DT):
                        ap = ap + c[t] * tg[t]
                    ns = [urow_v[ub + 1, pl.ds(D + 16 * t, 16)]
                          for t in range(DT)]
                    for kk in range(2, UROWS):
                        for t in range(DT):
                            ns[t] = ns[t] + urow_v[ub + kk,
                                                   pl.ds(D + 16 * t, 16)]
                    an = c[0] * ns[0]
                    for t in range(1, DT):
                        an = an + c[t] * ns[t]
                    accp = jnp.where(lanes == l, _hsum(ap, perms), accp)
                    accn = jnp.where(lanes == l, _hsum(an, perms), accn)
                    return accp, accn

                accp, accn = lax.fori_loop(0, 16, item_body,
                                           (zero16, zero16))
                pos_v[pl.ds(jbase, 16)] = accp
                neg_v[pl.ds(jbase, 16)] = -accn
                return c3

            lax.fori_loop(0, CH // 16, group_body, 0)
            pltpu.sync_copy(pos_v, pos_hbm.at[pl.ds(base, CH)])
            pltpu.sync_copy(neg_v, neg_hbm.at[pl.ds(base, CH)])
            return carry

        lax.fori_loop(0, nch, chunk_body, 0)

    return k(tbl, cidx, uidx)


def _tc_loss(pos2d, neg2d):
    n = pos2d.shape[0] * pos2d.shape[1]

    def body(p_ref, n_ref, o_ref):
        def logsig(x):
            return jnp.minimum(x, 0.0) - jnp.log1p(jnp.exp(-jnp.abs(x)))

        tot = jnp.sum(logsig(p_ref[...]) + logsig(n_ref[...]))
        o_ref[0, 0] = -tot / n

    return pl.pallas_call(
        body,
        out_shape=jax.ShapeDtypeStruct((1, 1), jnp.float32),
        out_specs=pl.BlockSpec(memory_space=pltpu.SMEM),
    )(pos2d, neg2d)


@jax.jit
def kernel(center_words, target_words, negative_words, embedding_v, embedding_u):
    B = center_words.shape[0]
    cidx = center_words.reshape(B).astype(jnp.int32)
    uidx = jnp.concatenate(
        [target_words.astype(jnp.int32), negative_words.astype(jnp.int32)],
        axis=1).reshape(-1)
    tbl = _sc_merge(embedding_v, embedding_u)
    pos, neg = _sc_scores(cidx, uidx, tbl)
    loss = _tc_loss(pos.reshape(128, -1), neg.reshape(128, -1))
    return loss[0, 0]


# speedup vs baseline: 1.4933x; 1.4933x over previous
import functools

import jax
import jax.numpy as jnp
from jax import lax
from jax.experimental import pallas as pl
from jax.experimental.pallas import tpu as pltpu
from jax.experimental.pallas import tpu_sc as plsc

D = 64
W = 128
K = 20
UROWS = K + 1
NW = 32
CH = 32
DT = D // 16

_GDN = lax.GatherDimensionNumbers(
    offset_dims=(), collapsed_slice_dims=(0,), start_index_map=(0,))


def _reg_gather(v, idx):
    return lax.gather(v, idx[:, None], _GDN, (1,),
                      mode=lax.GatherScatterMode.PROMISE_IN_BOUNDS)


def _hsum(v, perms):
    for p in perms:
        v = v + _reg_gather(v, p)
    return v


def _sc_scores(cidx, uidx, emb_v, emb_u):
    B = cidx.shape[0]
    per_w = B // NW
    nch = per_w // CH
    mesh = plsc.VectorSubcoreMesh(core_axis_name="c", subcore_axis_name="s",
                                  num_cores=2, num_subcores=16)

    @functools.partial(
        pl.kernel,
        out_type=[jax.ShapeDtypeStruct((B,), jnp.float32),
                  jax.ShapeDtypeStruct((B,), jnp.float32)],
        mesh=mesh,
        scratch_types=[
            pltpu.VMEM((CH,), jnp.int32),
            pltpu.VMEM((CH * UROWS,), jnp.int32),
            pltpu.VMEM((CH, W), jnp.float32),
            pltpu.VMEM((CH * UROWS, W), jnp.float32),
            pltpu.VMEM((CH,), jnp.float32),
            pltpu.VMEM((CH,), jnp.float32),
            pltpu.SemaphoreType.DMA,
        ],
        compiler_params=pltpu.CompilerParams(use_tc_tiling_on_sc=True),
    )
    def k(v_hbm, u_hbm, cidx_hbm, uidx_hbm, pos_hbm, neg_hbm,
          cidx_v, uidx_v, crow_v, urow_v, pos_v, neg_v, sem):
        wid = lax.axis_index("s") * 2 + lax.axis_index("c")
        base_w = wid * per_w
        lanes = lax.iota(jnp.int32, 16)
        perms = [lanes ^ s for s in (1, 2, 4, 8)]

        def chunk_body(ch, carry):
            base = base_w + ch * CH
            pltpu.sync_copy(cidx_hbm.at[pl.ds(base, CH)], cidx_v)
            pltpu.sync_copy(uidx_hbm.at[pl.ds(base * UROWS, CH * UROWS)],
                            uidx_v)
            copies = [pltpu.async_copy(v_hbm.at[cidx_v], crow_v, sem)]
            for r in range(UROWS):
                copies.append(pltpu.async_copy(
                    u_hbm.at[uidx_v.at[pl.ds(r * CH, CH)]],
                    urow_v.at[pl.ds(r * CH, CH)], sem))
            for cpy in copies:
                cpy.wait()

            zero16 = jnp.zeros((16,), jnp.float32)

            def group_body(g, c3):
                jbase = g * 16

                def item_body(l, acc):
                    accp, accn = acc
                    j = jbase + l
                    c = [crow_v[j, pl.ds(16 * t, 16)] for t in range(DT)]
                    ub = j * UROWS
                    tg = [urow_v[ub, pl.ds(16 * t, 16)]
                          for t in range(DT)]
                    ap = c[0] * tg[0]
                    for t in range(1, DT):
                        ap = ap + c[t] * tg[t]
                    ns = [urow_v[ub + 1, pl.ds(16 * t, 16)]
                          for t in range(DT)]
                    for kk in range(2, UROWS):
                        for t in range(DT):
                            ns[t] = ns[t] + urow_v[ub + kk,
                                                   pl.ds(16 * t, 16)]
                    an = c[0] * ns[0]
                    for t in range(1, DT):
                        an = an + c[t] * ns[t]
                    accp = jnp.where(lanes == l, _hsum(ap, perms), accp)
                    accn = jnp.where(lanes == l, _hsum(an, perms), accn)
                    return accp, accn

                accp, accn = lax.fori_loop(0, 16, item_body,
                                           (zero16, zero16))
                pos_v[pl.ds(jbase, 16)] = accp
                neg_v[pl.ds(jbase, 16)] = -accn
                return c3

            lax.fori_loop(0, CH // 16, group_body, 0)
            pltpu.sync_copy(pos_v, pos_hbm.at[pl.ds(base, CH)])
            pltpu.sync_copy(neg_v, neg_hbm.at[pl.ds(base, CH)])
            return carry

        lax.fori_loop(0, nch, chunk_body, 0)

    return k(emb_v, emb_u, cidx, uidx)


def _tc_loss(pos2d, neg2d):
    n = pos2d.shape[0] * pos2d.shape[1]

    def body(p_ref, n_ref, o_ref):
        def logsig(x):
            return jnp.minimum(x, 0.0) - jnp.log1p(jnp.exp(-jnp.abs(x)))

        tot = jnp.sum(logsig(p_ref[...]) + logsig(n_ref[...]))
        o_ref[0, 0] = -tot / n

    return pl.pallas_call(
        body,
        out_shape=jax.ShapeDtypeStruct((1, 1), jnp.float32),
        out_specs=pl.BlockSpec(memory_space=pltpu.SMEM),
    )(pos2d, neg2d)


@jax.jit
def kernel(center_words, target_words, negative_words, embedding_v, embedding_u):
    B = center_words.shape[0]
    cidx = center_words.reshape(B).astype(jnp.int32)
    uidx = jnp.concatenate(
        [target_words.astype(jnp.int32), negative_words.astype(jnp.int32)],
        axis=1).reshape(-1)
    vpad = jnp.pad(embedding_v, ((0, 0), (0, W - D)))
    upad = jnp.pad(embedding_u, ((0, 0), (0, W - D)))
    pos, neg = _sc_scores(cidx, uidx, vpad, upad)
    loss = _tc_loss(pos.reshape(128, -1), neg.reshape(128, -1))
    return loss[0, 0]


# speedup vs baseline: 1.5348x; 1.0278x over previous
import functools

import jax
import jax.numpy as jnp
from jax import lax
from jax.experimental import pallas as pl
from jax.experimental.pallas import tpu as pltpu
from jax.experimental.pallas import tpu_sc as plsc

D = 64
W = 128
K = 20
UROWS = K + 1
NW = 32
CH = 16
DT = D // 16

_GDN = lax.GatherDimensionNumbers(
    offset_dims=(), collapsed_slice_dims=(0,), start_index_map=(0,))


def _reg_gather(v, idx):
    return lax.gather(v, idx[:, None], _GDN, (1,),
                      mode=lax.GatherScatterMode.PROMISE_IN_BOUNDS)


def _hsum(v, perms):
    for p in perms:
        v = v + _reg_gather(v, p)
    return v


def _sc_scores(cidx, uidx, emb_v, emb_u):
    B = cidx.shape[0]
    per_w = B // NW
    nch = per_w // CH
    mesh = plsc.VectorSubcoreMesh(core_axis_name="c", subcore_axis_name="s",
                                  num_cores=2, num_subcores=16)

    @functools.partial(
        pl.kernel,
        out_type=[jax.ShapeDtypeStruct((B,), jnp.float32),
                  jax.ShapeDtypeStruct((B,), jnp.float32)],
        mesh=mesh,
        scratch_types=[
            pltpu.VMEM((CH,), jnp.int32),
            pltpu.VMEM((CH * UROWS,), jnp.int32),
            pltpu.VMEM((CH * 8, D), jnp.float32),
            pltpu.VMEM((CH * UROWS, W), jnp.float32),
            pltpu.VMEM((CH,), jnp.float32),
            pltpu.VMEM((CH,), jnp.float32),
            pltpu.SemaphoreType.DMA,
        ],
        compiler_params=pltpu.CompilerParams(use_tc_tiling_on_sc=True),
    )
    def k(v_hbm, u_hbm, cidx_hbm, uidx_hbm, pos_hbm, neg_hbm,
          cidx_v, uidx_v, crow_v, urow_v, pos_v, neg_v, sem):
        wid = lax.axis_index("s") * 2 + lax.axis_index("c")
        base_w = wid * per_w
        lanes = lax.iota(jnp.int32, 16)
        perms = [lanes ^ s for s in (1, 2, 4, 8)]

        def chunk_body(ch, carry):
            base = base_w + ch * CH
            pltpu.sync_copy(cidx_hbm.at[pl.ds(base, CH)], cidx_v)
            pltpu.sync_copy(uidx_hbm.at[pl.ds(base * UROWS, CH * UROWS)],
                            uidx_v)
            copies = []
            cvecs = [cidx_v[pl.ds(16 * g, 16)] for g in range(CH // 16)]
            for g in range(CH // 16):
                for l in range(16):
                    j = 16 * g + l
                    s = cvecs[g][l]
                    copies.append(pltpu.async_copy(
                        v_hbm.at[pl.ds((s // 8) * 8, 8), :],
                        crow_v.at[pl.ds(j * 8, 8), :], sem))
            for r in range(UROWS):
                copies.append(pltpu.async_copy(
                    u_hbm.at[uidx_v.at[pl.ds(r * CH, CH)]],
                    urow_v.at[pl.ds(r * CH, CH)], sem))
            for cpy in copies:
                cpy.wait()

            zero16 = jnp.zeros((16,), jnp.float32)

            for g in range(CH // 16):
                jbase = g * 16
                accp = zero16
                accn = zero16
                for l in range(16):
                    j = jbase + l
                    s = cvecs[g][l]
                    cr = j * 8 + s % 8
                    c = [crow_v[cr, pl.ds(16 * t, 16)] for t in range(DT)]
                    ub = j * UROWS
                    tg = [urow_v[ub, pl.ds(16 * t, 16)]
                          for t in range(DT)]
                    ap = c[0] * tg[0]
                    for t in range(1, DT):
                        ap = ap + c[t] * tg[t]
                    ns = [urow_v[ub + 1, pl.ds(16 * t, 16)]
                          for t in range(DT)]
                    for kk in range(2, UROWS):
                        for t in range(DT):
                            ns[t] = ns[t] + urow_v[ub + kk,
                                                   pl.ds(16 * t, 16)]
                    an = c[0] * ns[0]
                    for t in range(1, DT):
                        an = an + c[t] * ns[t]
                    accp = jnp.where(lanes == l, _hsum(ap, perms), accp)
                    accn = jnp.where(lanes == l, _hsum(an, perms), accn)
                pos_v[pl.ds(jbase, 16)] = accp
                neg_v[pl.ds(jbase, 16)] = -accn

            pltpu.sync_copy(pos_v, pos_hbm.at[pl.ds(base, CH)])
            pltpu.sync_copy(neg_v, neg_hbm.at[pl.ds(base, CH)])
            return carry

        lax.fori_loop(0, nch, chunk_body, 0)

    return k(emb_v, emb_u, cidx, uidx)


def _tc_loss(pos2d, neg2d):
    n = pos2d.shape[0] * pos2d.shape[1]

    def body(p_ref, n_ref, o_ref):
        def logsig(x):
            return jnp.minimum(x, 0.0) - jnp.log1p(jnp.exp(-jnp.abs(x)))

        tot = jnp.sum(logsig(p_ref[...]) + logsig(n_ref[...]))
        o_ref[0, 0] = -tot / n

    return pl.pallas_call(
        body,
        out_shape=jax.ShapeDtypeStruct((1, 1), jnp.float32),
        out_specs=pl.BlockSpec(memory_space=pltpu.SMEM),
    )(pos2d, neg2d)


@jax.jit
def kernel(center_words, target_words, negative_words, embedding_v, embedding_u):
    B = center_words.shape[0]
    cidx = center_words.reshape(B).astype(jnp.int32)
    uidx = jnp.concatenate(
        [target_words.astype(jnp.int32), negative_words.astype(jnp.int32)],
        axis=1).reshape(-1)
    upad = jnp.pad(embedding_u, ((0, 0), (0, W - D)))
    pos, neg = _sc_scores(cidx, uidx, embedding_v, upad)
    loss = _tc_loss(pos.reshape(128, -1), neg.reshape(128, -1))
    return loss[0, 0]
